# baseline (speedup 1.0000x reference)
import jax
import jax.numpy as jnp
from jax.experimental import pallas as pl

_B = 32
_C = 384
_G = 96
_GROUPBY = 4
_NSEL = 47


def _group_mask():
    key = jax.random.key(42)
    keys = jax.random.split(key, _B)
    notp = jnp.arange(1, _G, dtype=jnp.int32)
    chosen = jax.vmap(lambda k: jax.random.permutation(k, notp)[:_NSEL])(keys)
    mask = jnp.zeros((_B, _G), jnp.float32).at[:, 0].set(1.0)
    mask = mask.at[jnp.arange(_B)[:, None], chosen].set(1.0)
    return mask


def _mul_body(x_ref, m_ref, o_ref):
    o_ref[...] = x_ref[...] * m_ref[...]


def kernel(input):
    B, C, H, W = input.shape
    xt = jnp.transpose(input, (0, 2, 3, 1)).reshape(B, H * W, C)
    m = jnp.repeat(_group_mask(), _GROUPBY, axis=1).reshape(B, 1, C)
    chunk = 784
    out = pl.pallas_call(
        _mul_body,
        grid=(B, H * W // chunk),
        in_specs=[
            pl.BlockSpec((1, chunk, C), lambda b, i: (b, i, 0)),
            pl.BlockSpec((1, 1, C), lambda b, i: (b, 0, 0)),
        ],
        out_specs=pl.BlockSpec((1, chunk, C), lambda b, i: (b, i, 0)),
        out_shape=jax.ShapeDtypeStruct((B, H * W, C), jnp.float32),
    )(xt, m)
    return jnp.transpose(out.reshape(B, H, W, C), (0, 3, 1, 2))


# speedup vs baseline: 1.3832x; 1.3832x over previous
import jax
import jax.numpy as jnp
from jax.experimental import pallas as pl

_B = 32
_C = 384
_G = 96
_GROUPBY = 4
_NSEL = 47


def _group_mask():
    key = jax.random.key(42)
    keys = jax.random.split(key, _B)
    notp = jnp.arange(1, _G, dtype=jnp.int32)
    chosen = jax.vmap(lambda k: jax.random.permutation(k, notp)[:_NSEL])(keys)
    mask = jnp.zeros((_B, _G), jnp.float32).at[:, 0].set(1.0)
    mask = mask.at[jnp.arange(_B)[:, None], chosen].set(1.0)
    return mask


def _mul_body(x_ref, m_ref, o_ref):
    o_ref[...] = x_ref[...] * m_ref[...]


def kernel(input):
    B, C, H, W = input.shape
    xt = jnp.transpose(input, (0, 2, 3, 1)).reshape(B, H * W, C)
    m = jnp.repeat(_group_mask(), _GROUPBY, axis=1).reshape(B, 1, C)
    bb = 2
    out = pl.pallas_call(
        _mul_body,
        grid=(B // bb,),
        in_specs=[
            pl.BlockSpec((bb, H * W, C), lambda b: (b, 0, 0)),
            pl.BlockSpec((bb, 1, C), lambda b: (b, 0, 0)),
        ],
        out_specs=pl.BlockSpec((bb, H * W, C), lambda b: (b, 0, 0)),
        out_shape=jax.ShapeDtypeStruct((B, H * W, C), jnp.float32),
    )(xt, m)
    return jnp.transpose(out.reshape(B, H, W, C), (0, 3, 1, 2))
